# baseline (speedup 1.0000x reference)
import jax
import jax.numpy as jnp
from jax import lax
from jax.experimental import pallas as pl
from jax.experimental.pallas import tpu as pltpu
from jax.experimental.pallas import tpu_sc as plsc

B, N, E = 2, 10000, 128
R = 8
NE = 320000

NC, NS, L = 2, 16, 16
ET = NE // NS
CE = 2048
NK = (ET + CE - 1) // CE
TAIL = ET - (NK - 1) * CE
CH = 128
ZROW = B * N
OHZ = 128

WS = 1000
NW = N // WS
AR = WS * R
APT = AR // NS
CR = 640
CPT = CR // NS


def _sc_body(embf, src, rel, dst, bidx, z32, onehot,
             acc_out, cnt_out,
             s_src, s_rel, s_dst, s_bidx, row_ch, g_ch, row2_ch, g2_ch,
             row_chB, g_chB, rows_v, rows_vB, zacc, sem, semB,
             acc_sp, cnt_sp):
    core = lax.axis_index("c")
    sub = lax.axis_index("s")

    pltpu.sync_copy(z32, zacc)
    base = sub * ET

    c0 = sub * CPT

    def zcnt(q, _):
        pltpu.sync_copy(zacc.at[pl.ds(0, 8)],
                        cnt_sp.at[pl.ds(c0 + q * 8, 8)])
        return 0

    lax.fori_loop(0, CPT // 8, zcnt, 0)

    for w in range(NW):
        wlo = w * WS

        z0 = sub * APT

        def zero(q, _, z0=z0):
            pltpu.sync_copy(zacc, acc_sp.at[pl.ds(z0 + q * 32, 32)])
            return 0

        lax.fori_loop(0, APT // 32, zero, 0)
        rem = APT - (APT // 32) * 32
        if rem:
            pltpu.sync_copy(zacc.at[pl.ds(0, rem)],
                            acc_sp.at[pl.ds(z0 + (APT // 32) * 32, rem)])
        plsc.subcore_barrier()

        def emit_idx(j, rbuf, gbuf, wlo=wlo):
            for v in range(CH // L):
                sl = pl.ds(j * CH + v * L, L)
                s_ = s_src[sl]
                r_ = s_rel[sl]
                d_ = s_dst[sl]
                b_ = s_bidx[sl]
                mkb = (b_ == core)
                mk = (mkb & (s_ >= wlo) & (s_ < wlo + WS)).astype(jnp.int32)
                csl = pl.ds(v * L, L)
                rbuf[csl] = mk * (r_ * WS + s_ - wlo)
                gbuf[csl] = mk * (d_ + N * core - ZROW) + ZROW

        def emit_cnt_idx(j):
            for v in range(CH // L):
                sl = pl.ds(j * CH + v * L, L)
                s_ = s_src[sl]
                r_ = s_rel[sl]
                b_ = s_bidx[sl]
                mki = (b_ == core).astype(jnp.int32)
                fl = (s_ << 3) + r_
                csl = pl.ds(v * L, L)
                row2_ch[csl] = mki * (fl >> 7)
                g2_ch[csl] = mki * ((fl & 127) - OHZ) + OHZ

        def scan(q, _, w=w):
            j0 = 2 * q
            j1 = 2 * q + 1
            emit_idx(j0, row_ch, g_ch)
            h0 = pltpu.async_copy(embf.at[g_ch], rows_v, sem)
            emit_idx(j1, row_chB, g_chB)
            h1 = pltpu.async_copy(embf.at[g_chB], rows_vB, semB)
            h0.wait()
            pltpu.sync_copy(rows_v, acc_sp.at[row_ch], add=True)
            h1.wait()
            pltpu.sync_copy(rows_vB, acc_sp.at[row_chB], add=True)
            if w == 0:
                emit_cnt_idx(j0)
                pltpu.async_copy(onehot.at[g2_ch], rows_v, sem).wait()
                pltpu.sync_copy(rows_v, cnt_sp.at[row2_ch], add=True)
                emit_cnt_idx(j1)
                pltpu.async_copy(onehot.at[g2_ch], rows_v, sem).wait()
                pltpu.sync_copy(rows_v, cnt_sp.at[row2_ch], add=True)
            return 0

        def chunk(k, _, scan=scan):
            off = base + k * CE
            pltpu.sync_copy(src.at[pl.ds(off, CE)], s_src)
            pltpu.sync_copy(rel.at[pl.ds(off, CE)], s_rel)
            pltpu.sync_copy(dst.at[pl.ds(off, CE)], s_dst)
            pltpu.sync_copy(bidx.at[pl.ds(off, CE)], s_bidx)
            lax.fori_loop(0, CE // CH // 2, scan, 0)
            return 0

        lax.fori_loop(0, NK - 1, chunk, 0)

        off = base + (NK - 1) * CE
        pltpu.sync_copy(src.at[pl.ds(off, TAIL)], s_src.at[pl.ds(0, TAIL)])
        pltpu.sync_copy(rel.at[pl.ds(off, TAIL)], s_rel.at[pl.ds(0, TAIL)])
        pltpu.sync_copy(dst.at[pl.ds(off, TAIL)], s_dst.at[pl.ds(0, TAIL)])
        pltpu.sync_copy(bidx.at[pl.ds(off, TAIL)], s_bidx.at[pl.ds(0, TAIL)])
        for t in range((CE - TAIL) // L):
            s_bidx[pl.ds(TAIL + t * L, L)] = jnp.full((L,), NC + 1, jnp.int32)
        lax.fori_loop(0, CE // CH // 2, scan, 0)

        plsc.subcore_barrier()

        pltpu.sync_copy(acc_sp.at[pl.ds(sub * APT, APT)],
                        acc_out.at[core, w, sub])
        if w == 0:
            pltpu.sync_copy(cnt_sp.at[pl.ds(sub * CPT, CPT)],
                            cnt_out.at[core, pl.ds(sub * CPT, CPT)])
        plsc.subcore_barrier()


def _sc_call(embf, src, rel, dst, bidx, onehot):
    z32 = jnp.zeros((32, E), jnp.float32)
    mesh = plsc.VectorSubcoreMesh(core_axis_name="c", subcore_axis_name="s",
                                  num_cores=NC)
    f = pl.kernel(
        _sc_body,
        out_type=(
            jax.ShapeDtypeStruct((B, NW, NS, APT, E), jnp.float32),
            jax.ShapeDtypeStruct((NC, CR, E), jnp.float32),
        ),
        mesh=mesh,
        compiler_params=pltpu.CompilerParams(needs_layout_passes=False),
        scratch_types=[
            pltpu.VMEM((CE,), jnp.int32),
            pltpu.VMEM((CE,), jnp.int32),
            pltpu.VMEM((CE,), jnp.int32),
            pltpu.VMEM((CE,), jnp.int32),
            pltpu.VMEM((CH,), jnp.int32),
            pltpu.VMEM((CH,), jnp.int32),
            pltpu.VMEM((CH,), jnp.int32),
            pltpu.VMEM((CH,), jnp.int32),
            pltpu.VMEM((CH,), jnp.int32),
            pltpu.VMEM((CH,), jnp.int32),
            pltpu.VMEM((CH, E), jnp.float32),
            pltpu.VMEM((CH, E), jnp.float32),
            pltpu.VMEM((32, E), jnp.float32),
            pltpu.SemaphoreType.DMA,
            pltpu.SemaphoreType.DMA,
            pltpu.VMEM_SHARED((AR, E), jnp.float32),
            pltpu.VMEM_SHARED((CR, E), jnp.float32),
        ],
    )
    return f(embf, src, rel, dst, bidx, z32, onehot)


NB = 2000
CB = NB * R // E


def _tc_body(emb_ref, acc_ref, cnt_ref, w_ref, e1_ref, m_ref, sel_ref,
             out_ref):
    acc = acc_ref[0]
    craw = cnt_ref[0, 0]
    recipraw = jnp.where(craw > 0, 1.0 / craw, 0.0)
    e1 = e1_ref[...]
    m = m_ref[...]
    ones16 = jnp.ones((16, E), jnp.float32)
    y = jnp.zeros((NB, E), jnp.float32)
    for r in range(R):
        a_r = lax.dot_general(recipraw, sel_ref[r],
                              dimension_numbers=(((1,), (0,)), ((), ())),
                              preferred_element_type=jnp.float32)
        c_r = lax.dot_general(e1, a_r,
                              dimension_numbers=(((1,), (0,)), ((), ())),
                              preferred_element_type=jnp.float32)
        d_r = lax.dot_general(c_r * m, ones16,
                              dimension_numbers=(((1,), (0,)), ((), ())),
                              preferred_element_type=jnp.float32)
        x_r = acc[r] * d_r
        y = y + lax.dot_general(x_r, w_ref[r],
                                dimension_numbers=(((1,), (1,)), ((), ())),
                                preferred_element_type=jnp.float32)
    out_ref[0] = jnp.maximum(emb_ref[0] + y, 0.0)


def _tc_call(emb, acc, cnt, weights, e1, m, sel):
    grid = (B, N // NB)
    return pl.pallas_call(
        _tc_body,
        grid=grid,
        in_specs=[
            pl.BlockSpec((1, NB, E), lambda b, i: (b, i, 0)),
            pl.BlockSpec((1, R, NB, E), lambda b, i: (b, 0, i, 0)),
            pl.BlockSpec((1, 1, CB, E), lambda b, i: (b, i, 0, 0)),
            pl.BlockSpec((R, E, E), lambda b, i: (0, 0, 0)),
            pl.BlockSpec((NB, CB), lambda b, i: (0, 0)),
            pl.BlockSpec((NB, 16), lambda b, i: (0, 0)),
            pl.BlockSpec((R, E, 16), lambda b, i: (0, 0, 0)),
        ],
        out_specs=pl.BlockSpec((1, NB, E), lambda b, i: (b, i, 0)),
        out_shape=jax.ShapeDtypeStruct((B, N, E), jnp.float32),
    )(emb, acc, cnt, weights, e1, m, sel)


def kernel(embeddings, relations, tokeys, toqueries, weights,
           src, rel, dst, bidx):
    src = src.astype(jnp.int32)
    rel = rel.astype(jnp.int32)
    dst = dst.astype(jnp.int32)
    bidx = bidx.astype(jnp.int32)
    embf = jnp.concatenate(
        [embeddings.reshape(B * N, E), jnp.zeros((8, E), jnp.float32)], axis=0)
    onehot = jnp.concatenate(
        [jnp.eye(E, dtype=jnp.float32), jnp.zeros((8, E), jnp.float32)],
        axis=0)
    acc, cnt = _sc_call(embf, src, rel, dst, bidx, onehot)
    acc4 = (acc.reshape(B, NW, R, WS, E)
            .transpose(0, 2, 1, 3, 4)
            .reshape(B, R, N, E))
    cntb = cnt[:, : R * N // E].reshape(B, N // NB, NB * R // E, E)
    sloc = jnp.arange(NB)
    e1 = (sloc[:, None] // 16 == jnp.arange(CB)[None, :]).astype(jnp.float32)
    m = (sloc[:, None] % 16 == jnp.arange(16)[None, :]).astype(jnp.float32)
    lane = jnp.arange(E)
    sel = (lane[None, :, None] ==
           (jnp.arange(16)[None, None, :] * R + jnp.arange(R)[:, None, None])
           ).astype(jnp.float32)
    return _tc_call(embeddings, acc4, cntb, weights, e1, m, sel)


# speedup vs baseline: 5.9544x; 5.9544x over previous
import jax
import jax.numpy as jnp
from jax import lax
from jax.experimental import pallas as pl
from jax.experimental.pallas import tpu as pltpu
from jax.experimental.pallas import tpu_sc as plsc

B, N, E = 2, 10000, 128
R = 8
NE = 320000

NC, NS, L = 2, 16, 16
ET = NE // NS
CE = 2048
NK = (ET + CE - 1) // CE
TAIL = ET - (NK - 1) * CE
CH = 128
ZROW = B * N
OHZ = 128

WS = 1000
NW = N // WS
AR = WS * R
APT = AR // NS
CR = 640
CPT = CR // NS


def _sc_body(embf, src, rel, dst, bidx, z32, onehot,
             acc_out, cnt_out,
             s_src, s_rel, s_dst, s_bidx, row_ch, g_ch, row2_ch, g2_ch,
             row_chB, g_chB, rows_v, rows_vB, zacc, sem, semB,
             acc_sp, cnt_sp):
    core = lax.axis_index("c")
    sub = lax.axis_index("s")

    pltpu.sync_copy(z32, zacc)
    base = sub * ET

    c0 = sub * CPT

    def zcnt(q, _):
        pltpu.sync_copy(zacc.at[pl.ds(0, 8)],
                        cnt_sp.at[pl.ds(c0 + q * 8, 8)])
        return 0

    lax.fori_loop(0, CPT // 8, zcnt, 0)

    iota = lax.iota(jnp.int32, L)
    zr8 = ZROW + (iota & 7)
    jv = [(v * 16 + sub * 512 + iota) & 4095 for v in range(CH // L)]
    jv2 = [(v * 16 + sub * 32 + iota) & 511 for v in range(CH // L)]

    for w in range(NW):
        wlo = w * WS

        z0 = sub * APT

        def zero(q, _, z0=z0):
            pltpu.sync_copy(zacc, acc_sp.at[pl.ds(z0 + q * 32, 32)])
            return 0

        lax.fori_loop(0, APT // 32, zero, 0)
        rem = APT - (APT // 32) * 32
        if rem:
            pltpu.sync_copy(zacc.at[pl.ds(0, rem)],
                            acc_sp.at[pl.ds(z0 + (APT // 32) * 32, rem)])
        plsc.subcore_barrier()

        def emit_idx(j, rbuf, gbuf, wlo=wlo):
            for v in range(CH // L):
                sl = pl.ds(j * CH + v * L, L)
                s_ = s_src[sl]
                r_ = s_rel[sl]
                d_ = s_dst[sl]
                b_ = s_bidx[sl]
                mkb = (b_ == core)
                mk = (mkb & (s_ >= wlo) & (s_ < wlo + WS)).astype(jnp.int32)
                csl = pl.ds(v * L, L)
                rbuf[csl] = mk * (r_ * WS + s_ - wlo - jv[v]) + jv[v]
                gbuf[csl] = mk * (d_ + N * core - zr8) + zr8

        def emit_cnt_idx(j):
            for v in range(CH // L):
                sl = pl.ds(j * CH + v * L, L)
                s_ = s_src[sl]
                r_ = s_rel[sl]
                b_ = s_bidx[sl]
                mki = (b_ == core).astype(jnp.int32)
                fl = (s_ << 3) + r_
                csl = pl.ds(v * L, L)
                row2_ch[csl] = mki * ((fl >> 7) - jv2[v]) + jv2[v]
                g2_ch[csl] = mki * ((fl & 127) - OHZ) + OHZ

        def scan(q, _, w=w):
            j0 = 2 * q
            j1 = 2 * q + 1
            emit_idx(j0, row_ch, g_ch)
            h0 = pltpu.async_copy(embf.at[g_ch], rows_v, sem)
            emit_idx(j1, row_chB, g_chB)
            h1 = pltpu.async_copy(embf.at[g_chB], rows_vB, semB)
            h0.wait()
            pltpu.sync_copy(rows_v, acc_sp.at[row_ch], add=True)
            h1.wait()
            pltpu.sync_copy(rows_vB, acc_sp.at[row_chB], add=True)
            if w == 0:
                emit_cnt_idx(j0)
                pltpu.async_copy(onehot.at[g2_ch], rows_v, sem).wait()
                pltpu.sync_copy(rows_v, cnt_sp.at[row2_ch], add=True)
                emit_cnt_idx(j1)
                pltpu.async_copy(onehot.at[g2_ch], rows_v, sem).wait()
                pltpu.sync_copy(rows_v, cnt_sp.at[row2_ch], add=True)
            return 0

        def chunk(k, _, scan=scan):
            off = base + k * CE
            pltpu.sync_copy(src.at[pl.ds(off, CE)], s_src)
            pltpu.sync_copy(rel.at[pl.ds(off, CE)], s_rel)
            pltpu.sync_copy(dst.at[pl.ds(off, CE)], s_dst)
            pltpu.sync_copy(bidx.at[pl.ds(off, CE)], s_bidx)
            lax.fori_loop(0, CE // CH // 2, scan, 0)
            return 0

        lax.fori_loop(0, NK - 1, chunk, 0)

        off = base + (NK - 1) * CE
        pltpu.sync_copy(src.at[pl.ds(off, TAIL)], s_src.at[pl.ds(0, TAIL)])
        pltpu.sync_copy(rel.at[pl.ds(off, TAIL)], s_rel.at[pl.ds(0, TAIL)])
        pltpu.sync_copy(dst.at[pl.ds(off, TAIL)], s_dst.at[pl.ds(0, TAIL)])
        pltpu.sync_copy(bidx.at[pl.ds(off, TAIL)], s_bidx.at[pl.ds(0, TAIL)])
        for t in range((CE - TAIL) // L):
            s_bidx[pl.ds(TAIL + t * L, L)] = jnp.full((L,), NC + 1, jnp.int32)
        lax.fori_loop(0, CE // CH // 2, scan, 0)

        plsc.subcore_barrier()

        pltpu.sync_copy(acc_sp.at[pl.ds(sub * APT, APT)],
                        acc_out.at[core, w, sub])
        if w == 0:
            pltpu.sync_copy(cnt_sp.at[pl.ds(sub * CPT, CPT)],
                            cnt_out.at[core, pl.ds(sub * CPT, CPT)])
        plsc.subcore_barrier()


def _sc_call(embf, src, rel, dst, bidx, onehot):
    z32 = jnp.zeros((32, E), jnp.float32)
    mesh = plsc.VectorSubcoreMesh(core_axis_name="c", subcore_axis_name="s",
                                  num_cores=NC)
    f = pl.kernel(
        _sc_body,
        out_type=(
            jax.ShapeDtypeStruct((B, NW, NS, APT, E), jnp.float32),
            jax.ShapeDtypeStruct((NC, CR, E), jnp.float32),
        ),
        mesh=mesh,
        compiler_params=pltpu.CompilerParams(needs_layout_passes=False),
        scratch_types=[
            pltpu.VMEM((CE,), jnp.int32),
            pltpu.VMEM((CE,), jnp.int32),
            pltpu.VMEM((CE,), jnp.int32),
            pltpu.VMEM((CE,), jnp.int32),
            pltpu.VMEM((CH,), jnp.int32),
            pltpu.VMEM((CH,), jnp.int32),
            pltpu.VMEM((CH,), jnp.int32),
            pltpu.VMEM((CH,), jnp.int32),
            pltpu.VMEM((CH,), jnp.int32),
            pltpu.VMEM((CH,), jnp.int32),
            pltpu.VMEM((CH, E), jnp.float32),
            pltpu.VMEM((CH, E), jnp.float32),
            pltpu.VMEM((32, E), jnp.float32),
            pltpu.SemaphoreType.DMA,
            pltpu.SemaphoreType.DMA,
            pltpu.VMEM_SHARED((AR, E), jnp.float32),
            pltpu.VMEM_SHARED((CR, E), jnp.float32),
        ],
    )
    return f(embf, src, rel, dst, bidx, z32, onehot)


NB = 2000
CB = NB * R // E


def _tc_body(emb_ref, acc_ref, cnt_ref, w_ref, e1_ref, m_ref, sel_ref,
             out_ref):
    acc = acc_ref[0]
    craw = cnt_ref[0, 0]
    recipraw = jnp.where(craw > 0, 1.0 / craw, 0.0)
    e1 = e1_ref[...]
    m = m_ref[...]
    ones16 = jnp.ones((16, E), jnp.float32)
    y = jnp.zeros((NB, E), jnp.float32)
    for r in range(R):
        a_r = lax.dot_general(recipraw, sel_ref[r],
                              dimension_numbers=(((1,), (0,)), ((), ())),
                              preferred_element_type=jnp.float32)
        c_r = lax.dot_general(e1, a_r,
                              dimension_numbers=(((1,), (0,)), ((), ())),
                              preferred_element_type=jnp.float32)
        d_r = lax.dot_general(c_r * m, ones16,
                              dimension_numbers=(((1,), (0,)), ((), ())),
                              preferred_element_type=jnp.float32)
        x_r = acc[r] * d_r
        y = y + lax.dot_general(x_r, w_ref[r],
                                dimension_numbers=(((1,), (1,)), ((), ())),
                                preferred_element_type=jnp.float32)
    out_ref[0] = jnp.maximum(emb_ref[0] + y, 0.0)


def _tc_call(emb, acc, cnt, weights, e1, m, sel):
    grid = (B, N // NB)
    return pl.pallas_call(
        _tc_body,
        grid=grid,
        in_specs=[
            pl.BlockSpec((1, NB, E), lambda b, i: (b, i, 0)),
            pl.BlockSpec((1, R, NB, E), lambda b, i: (b, 0, i, 0)),
            pl.BlockSpec((1, 1, CB, E), lambda b, i: (b, i, 0, 0)),
            pl.BlockSpec((R, E, E), lambda b, i: (0, 0, 0)),
            pl.BlockSpec((NB, CB), lambda b, i: (0, 0)),
            pl.BlockSpec((NB, 16), lambda b, i: (0, 0)),
            pl.BlockSpec((R, E, 16), lambda b, i: (0, 0, 0)),
        ],
        out_specs=pl.BlockSpec((1, NB, E), lambda b, i: (b, i, 0)),
        out_shape=jax.ShapeDtypeStruct((B, N, E), jnp.float32),
    )(emb, acc, cnt, weights, e1, m, sel)


def kernel(embeddings, relations, tokeys, toqueries, weights,
           src, rel, dst, bidx):
    src = src.astype(jnp.int32)
    rel = rel.astype(jnp.int32)
    dst = dst.astype(jnp.int32)
    bidx = bidx.astype(jnp.int32)
    embf = jnp.concatenate(
        [embeddings.reshape(B * N, E), jnp.zeros((8, E), jnp.float32)], axis=0)
    onehot = jnp.concatenate(
        [jnp.eye(E, dtype=jnp.float32), jnp.zeros((8, E), jnp.float32)],
        axis=0)
    acc, cnt = _sc_call(embf, src, rel, dst, bidx, onehot)
    acc4 = (acc.reshape(B, NW, R, WS, E)
            .transpose(0, 2, 1, 3, 4)
            .reshape(B, R, N, E))
    cntb = cnt[:, : R * N // E].reshape(B, N // NB, NB * R // E, E)
    sloc = jnp.arange(NB)
    e1 = (sloc[:, None] // 16 == jnp.arange(CB)[None, :]).astype(jnp.float32)
    m = (sloc[:, None] % 16 == jnp.arange(16)[None, :]).astype(jnp.float32)
    lane = jnp.arange(E)
    sel = (lane[None, :, None] ==
           (jnp.arange(16)[None, None, :] * R + jnp.arange(R)[:, None, None])
           ).astype(jnp.float32)
    return _tc_call(embeddings, acc4, cntb, weights, e1, m, sel)
